# baseline (speedup 1.0000x reference)
import jax
import jax.numpy as jnp
from jax.experimental import pallas as pl

_EPS = 1e-5
_BLOCK_S = 1024
_BLOCK_B = 2


def _embed_ln_body(x_ref, pos_ref, g_ref, b_ref, o_ref):
    v = x_ref[...] + pos_ref[...][None, :, :]
    mean = jnp.mean(v, axis=-1, keepdims=True)
    c = v - mean
    var = jnp.mean(c * c, axis=-1, keepdims=True)
    o_ref[...] = c * jax.lax.rsqrt(var + _EPS) * g_ref[...] + b_ref[...]


def kernel(x, pos_table, ln_gamma, ln_beta, batch_size):
    del batch_size
    B, S, D = x.shape
    g2 = ln_gamma.reshape(1, 1, D)
    b2 = ln_beta.reshape(1, 1, D)
    return pl.pallas_call(
        _embed_ln_body,
        grid=(S // _BLOCK_S, B // _BLOCK_B),
        in_specs=[
            pl.BlockSpec((_BLOCK_B, _BLOCK_S, D), lambda i, j: (j, i, 0)),
            pl.BlockSpec((_BLOCK_S, D), lambda i, j: (i, 0)),
            pl.BlockSpec((1, 1, D), lambda i, j: (0, 0, 0)),
            pl.BlockSpec((1, 1, D), lambda i, j: (0, 0, 0)),
        ],
        out_specs=pl.BlockSpec((_BLOCK_B, _BLOCK_S, D), lambda i, j: (j, i, 0)),
        out_shape=jax.ShapeDtypeStruct((B, S, D), x.dtype),
    )(x, pos_table, g2, b2)


# speedup vs baseline: 1.0478x; 1.0478x over previous
import jax
import jax.numpy as jnp
from jax.experimental import pallas as pl

_EPS = 1e-5
_BLOCK_S = 512


def _embed_ln_body(x_ref, pos_ref, g_ref, b_ref, o_ref):
    v = x_ref[...] + pos_ref[...][None, :, :]
    d = v.shape[-1]
    s1 = jnp.sum(v, axis=-1, keepdims=True)
    s2 = jnp.sum(v * v, axis=-1, keepdims=True)
    mean = s1 * (1.0 / d)
    var = s2 * (1.0 / d) - mean * mean
    inv = jax.lax.rsqrt(var + _EPS)
    scale = inv * g_ref[...]
    shift = b_ref[...] - mean * scale
    o_ref[...] = v * scale + shift


def kernel(x, pos_table, ln_gamma, ln_beta, batch_size):
    del batch_size
    B, S, D = x.shape
    g2 = ln_gamma.reshape(1, 1, D)
    b2 = ln_beta.reshape(1, 1, D)
    return pl.pallas_call(
        _embed_ln_body,
        grid=(S // _BLOCK_S,),
        in_specs=[
            pl.BlockSpec((B, _BLOCK_S, D), lambda i: (0, i, 0)),
            pl.BlockSpec((_BLOCK_S, D), lambda i: (i, 0)),
            pl.BlockSpec((1, 1, D), lambda i: (0, 0, 0)),
            pl.BlockSpec((1, 1, D), lambda i: (0, 0, 0)),
        ],
        out_specs=pl.BlockSpec((B, _BLOCK_S, D), lambda i: (0, i, 0)),
        out_shape=jax.ShapeDtypeStruct((B, S, D), x.dtype),
    )(x, pos_table, g2, b2)


# speedup vs baseline: 1.0540x; 1.0059x over previous
import jax
import jax.numpy as jnp
from jax.experimental import pallas as pl

_EPS = 1e-5
_BLOCK_S = 512


def _embed_ln_body(x_ref, pos_ref, g_ref, b_ref, o_ref):
    v = x_ref[...] + pos_ref[...][None, :, :]
    mean = jnp.mean(v, axis=-1, keepdims=True)
    c = v - mean
    var = jnp.mean(c * c, axis=-1, keepdims=True)
    o_ref[...] = c * jax.lax.rsqrt(var + _EPS) * g_ref[...] + b_ref[...]


def kernel(x, pos_table, ln_gamma, ln_beta, batch_size):
    del batch_size
    B, S, D = x.shape
    g2 = ln_gamma.reshape(1, 1, D)
    b2 = ln_beta.reshape(1, 1, D)
    return pl.pallas_call(
        _embed_ln_body,
        grid=(S // _BLOCK_S,),
        in_specs=[
            pl.BlockSpec((B, _BLOCK_S, D), lambda i: (0, i, 0)),
            pl.BlockSpec((_BLOCK_S, D), lambda i: (i, 0)),
            pl.BlockSpec((1, 1, D), lambda i: (0, 0, 0)),
            pl.BlockSpec((1, 1, D), lambda i: (0, 0, 0)),
        ],
        out_specs=pl.BlockSpec((B, _BLOCK_S, D), lambda i: (0, i, 0)),
        out_shape=jax.ShapeDtypeStruct((B, S, D), x.dtype),
    )(x, pos_table, g2, b2)
